# Initial kernel scaffold; baseline (speedup 1.0000x reference)
import jax
import jax.numpy as jnp
from jax.experimental import pallas as pl


def kernel(x, edge_index, W1, b1, W2, b2):
    raise NotImplementedError("write your pallas kernel here")


# speedup vs baseline: 3.5321x; 3.5321x over previous
import functools

import jax
import jax.numpy as jnp
from jax import lax
from jax.experimental import pallas as pl
from jax.experimental.pallas import tpu as pltpu
from jax.experimental.pallas import tpu_sc as plsc

N = 10000
E = 160000
F = 256
C = 128

_BM = 1000
_CH = 80
_NSUB = 16
_EPW = E // _NSUB
_NCH = _EPW // _CH
_RPW = 640
_NPAD = _RPW * _NSUB


def _mm1(x, w1t):
    def body(x_ref, w_ref, oa_ref, ob_ref):
        y = jnp.dot(x_ref[...], w_ref[...], preferred_element_type=jnp.float32)
        oa_ref[...] = y[:, :128]
        ob_ref[...] = y[:, 128:]

    return pl.pallas_call(
        body,
        grid=(N // _BM,),
        in_specs=[
            pl.BlockSpec((_BM, F), lambda i: (i, 0)),
            pl.BlockSpec((F, F), lambda i: (0, 0)),
        ],
        out_specs=[
            pl.BlockSpec((_BM, 128), lambda i: (i, 0)),
            pl.BlockSpec((_BM, 128), lambda i: (i, 0)),
        ],
        out_shape=[jax.ShapeDtypeStruct((N, 128), jnp.float32)] * 2,
    )(x, w1t)


def _mm2(agg1a, agg1b, b1r, w2t):
    def body(aa_ref, ab_ref, b_ref, w_ref, o_ref):
        ha = jnp.maximum(aa_ref[...] + b_ref[:, :128], 0.0)
        hb = jnp.maximum(ab_ref[...] + b_ref[:, 128:], 0.0)
        y = jnp.dot(ha, w_ref[:128, :], preferred_element_type=jnp.float32)
        y = y + jnp.dot(hb, w_ref[128:, :], preferred_element_type=jnp.float32)
        o_ref[...] = y

    return pl.pallas_call(
        body,
        grid=(N // _BM,),
        in_specs=[
            pl.BlockSpec((_BM, 128), lambda i: (i, 0)),
            pl.BlockSpec((_BM, 128), lambda i: (i, 0)),
            pl.BlockSpec((1, F), lambda i: (0, 0)),
            pl.BlockSpec((F, C), lambda i: (0, 0)),
        ],
        out_specs=pl.BlockSpec((_BM, C), lambda i: (i, 0)),
        out_shape=jax.ShapeDtypeStruct((N, C), jnp.float32),
    )(agg1a, agg1b, b1r, w2t)


def _bias_relu(p0, p1, b2r):
    def body(p0_ref, p1_ref, b_ref, o_ref):
        o_ref[...] = jnp.maximum(p0_ref[...] + p1_ref[...] + b_ref[...], 0.0)

    return pl.pallas_call(
        body,
        grid=(N // _BM,),
        in_specs=[
            pl.BlockSpec((_BM, C), lambda i: (i, 0)),
            pl.BlockSpec((_BM, C), lambda i: (i, 0)),
            pl.BlockSpec((1, C), lambda i: (0, 0)),
        ],
        out_specs=pl.BlockSpec((_BM, C), lambda i: (i, 0)),
        out_shape=jax.ShapeDtypeStruct((N, C), jnp.float32),
    )(p0, p1, b2r)


def _writeout(acc, out, s):
    @pl.when(s < _NSUB - 1)
    def _():
        pltpu.sync_copy(acc.at[pl.ds(s * _RPW, _RPW)], out.at[pl.ds(s * _RPW, _RPW)])

    @pl.when(s == _NSUB - 1)
    def _():
        last = (_NSUB - 1) * _RPW
        pltpu.sync_copy(acc.at[pl.ds(last, N - last)], out.at[pl.ds(last, N - last)])


def _segsum_featsplit(t0, t1, src, dst, zrows):
    mesh = plsc.VectorSubcoreMesh(core_axis_name="c", subcore_axis_name="s")

    @functools.partial(
        pl.kernel,
        out_type=[jax.ShapeDtypeStruct((N, 128), jnp.float32)] * 2,
        mesh=mesh,
        scratch_types=[
            pltpu.VMEM((_CH,), jnp.int32),
            pltpu.VMEM((_CH,), jnp.int32),
            pltpu.VMEM((_CH, 128), jnp.float32),
            pltpu.VMEM_SHARED((_NPAD, 128), jnp.float32),
        ],
    )
    def k(t0_h, t1_h, src_h, dst_h, z_h, o0_h, o1_h, src_v, dst_v, rows_v, acc):
        c = lax.axis_index("c")
        s = lax.axis_index("s")
        pltpu.sync_copy(z_h, acc.at[pl.ds(s * _RPW, _RPW)])
        plsc.subcore_barrier()

        def run(tbl, out):
            @pl.loop(0, _NCH)
            def _(i):
                off = s * _EPW + i * _CH
                pltpu.sync_copy(src_h.at[pl.ds(off, _CH)], src_v)
                pltpu.sync_copy(dst_h.at[pl.ds(off, _CH)], dst_v)
                pltpu.sync_copy(tbl.at[src_v], rows_v)
                pltpu.sync_copy(rows_v, acc.at[dst_v], add=True)

            plsc.subcore_barrier()
            _writeout(acc, out, s)

        @pl.when(c == 0)
        def _():
            run(t0_h, o0_h)

        @pl.when(c == 1)
        def _():
            run(t1_h, o1_h)

    return k(t0, t1, src, dst, zrows)


_CH2 = 40
_EPW2 = E // (2 * _NSUB)


def _segsum_edgesplit(t, src, dst, zrows):
    mesh = plsc.VectorSubcoreMesh(core_axis_name="c", subcore_axis_name="s")

    @functools.partial(
        pl.kernel,
        out_type=[jax.ShapeDtypeStruct((N, C), jnp.float32)] * 2,
        mesh=mesh,
        scratch_types=[
            pltpu.VMEM((_CH2,), jnp.int32),
            pltpu.VMEM((_CH2,), jnp.int32),
            pltpu.VMEM((_CH2, C), jnp.float32),
            pltpu.VMEM_SHARED((_NPAD, C), jnp.float32),
        ],
    )
    def k(t_h, src_h, dst_h, z_h, o0_h, o1_h, src_v, dst_v, rows_v, acc):
        c = lax.axis_index("c")
        s = lax.axis_index("s")
        pltpu.sync_copy(z_h, acc.at[pl.ds(s * _RPW, _RPW)])
        plsc.subcore_barrier()
        base = (c * _NSUB + s) * _EPW2

        @pl.loop(0, _EPW2 // _CH2)
        def _(i):
            off = base + i * _CH2
            pltpu.sync_copy(src_h.at[pl.ds(off, _CH2)], src_v)
            pltpu.sync_copy(dst_h.at[pl.ds(off, _CH2)], dst_v)
            pltpu.sync_copy(t_h.at[src_v], rows_v)
            pltpu.sync_copy(rows_v, acc.at[dst_v], add=True)

        plsc.subcore_barrier()

        @pl.when(c == 0)
        def _():
            _writeout(acc, o0_h, s)

        @pl.when(c == 1)
        def _():
            _writeout(acc, o1_h, s)

    return k(t, src, dst, zrows)


def kernel(x, edge_index, W1, b1, W2, b2):
    ei = edge_index.astype(jnp.int32)
    src = ei[0]
    dst = ei[1]
    w1t = W1.T
    w2t = W2.T
    b1r = b1.reshape(1, F)
    b2r = b2.reshape(1, C)
    zrows = jnp.zeros((_RPW, 128), jnp.float32)

    y1a, y1b = _mm1(x, w1t)
    agg1a, agg1b = _segsum_featsplit(y1a, y1b, src, dst, zrows)
    y2 = _mm2(agg1a, agg1b, b1r, w2t)
    p0, p1 = _segsum_edgesplit(y2, src, dst, zrows)
    return _bias_relu(p0, p1, b2r)
